# baseline (speedup 1.0000x reference)
import functools

import jax
import jax.numpy as jnp
from jax.experimental import pallas as pl
from jax.experimental.pallas import tpu as pltpu

_B, _N, _C = 16, 5000, 200
_BN = 1000
_NBLK = _N // _BN
_GRID = _B * _NBLK


def _body(vd_ref, off_ref, logits_ref, center_ref, width_ref,
          scores_ref, labels_ref, t1_ref, t2_ref, mask_ref):
    g = pl.program_id(0)
    b = g // _NBLK

    x = logits_ref[0]
    m = jnp.max(x, axis=1)
    ids = jax.lax.broadcasted_iota(jnp.int32, x.shape, 1)
    lbl = jnp.min(jnp.where(x == m[:, None], ids, _C), axis=1)
    scores_ref[0, 0] = jax.nn.sigmoid(m)
    labels_ref[0, 0] = lbl

    off = off_ref[b]
    vd = vd_ref[b]
    c = center_ref[0, 0]
    half_w = 0.5 * jnp.exp(width_ref[0, 0])
    t1 = jnp.clip(c - half_w + off, 0.0, vd)
    t2 = jnp.clip(c + half_w + off, 0.0, vd)
    t1_ref[0, 0] = t1
    t2_ref[0, 0] = t2
    mask_ref[0, 0] = ((t2 - t1) > 0.05).astype(jnp.int32)


@jax.jit
def kernel(pred_logits, pred_segments, video_durations, feature_durations,
           strides, offsets):
    del feature_durations, strides
    logits3 = pred_logits.reshape(_GRID, _BN, _C)
    center3 = pred_segments[:, :, 0].reshape(_GRID, 1, _BN)
    width3 = pred_segments[:, :, 1].reshape(_GRID, 1, _BN)

    row_spec = pl.BlockSpec((1, 1, _BN), lambda g: (g, 0, 0))
    smem_spec = pl.BlockSpec(memory_space=pltpu.SMEM)
    out_sds = jax.ShapeDtypeStruct((_GRID, 1, _BN), jnp.float32)
    out_ids = jax.ShapeDtypeStruct((_GRID, 1, _BN), jnp.int32)

    scores, labels, t1, t2, mask = pl.pallas_call(
        _body,
        grid=(_GRID,),
        in_specs=[
            smem_spec,
            smem_spec,
            pl.BlockSpec((1, _BN, _C), lambda g: (g, 0, 0)),
            row_spec,
            row_spec,
        ],
        out_specs=[row_spec] * 5,
        out_shape=[out_sds, out_ids, out_sds, out_sds, out_ids],
    )(video_durations, offsets, logits3, center3, width3)

    scores = scores.reshape(_B, _N)
    labels = labels.reshape(_B, _N)
    segments = jnp.stack([t1.reshape(_B, _N), t2.reshape(_B, _N)], axis=-1)
    valid_mask = mask.reshape(_B, _N).astype(bool)
    return scores, labels, segments, valid_mask
